# baseline (speedup 1.0000x reference)
import functools

import jax
import jax.numpy as jnp
from jax import lax
from jax.experimental import pallas as pl
from jax.experimental.pallas import tpu as pltpu
from jax.experimental.pallas import tpu_sc as plsc

_VOCAB = 32000
_HID = 768
_B = 4
_S = 2048
_EPS = 1e-12
_NT = _B * _S
_NW = 32
_TPW = _NT // _NW
_C = 16
_NCHUNK = _TPW // _C
_G = _HID // 16
_SB = 16


def _rsqrt16(v):
    i = plsc.bitcast(v, jnp.int32)
    y = plsc.bitcast(jnp.int32(0x5F3759DF) - (i >> 1), jnp.float32)
    h = v * 0.5
    y = y * (1.5 - h * y * y)
    y = y * (1.5 - h * y * y)
    return y


def _tile_body(ids_hbm, wemb_hbm, pemb_hbm, gamma_hbm, beta_hbm, out_hbm,
               idx_all, rows_a, rows_b, pos_a, pos_b, o_a, o_b,
               gamma_v, beta_v, gsem_a, gsem_b, psem_a, psem_b,
               osem_a, osem_b, prosem):
    wid = lax.axis_index("s") * 2 + lax.axis_index("c")
    base = wid * _TPW
    s0 = base % _S

    pltpu.sync_copy(ids_hbm.at[pl.ds(base, _TPW)], idx_all)
    pltpu.async_copy(gamma_hbm, gamma_v, prosem)
    pltpu.async_copy(beta_hbm, beta_v, prosem)

    bufs = (
        (rows_a, pos_a, o_a, gsem_a, psem_a, osem_a),
        (rows_b, pos_b, o_b, gsem_b, psem_b, osem_b),
    )

    def issue_loads(ci, buf):
        rows_v, pos_v, _, gsem, psem, _ = buf
        pltpu.async_copy(wemb_hbm.at[idx_all.at[pl.ds(ci * _C, _C)]],
                         rows_v, gsem)
        pltpu.async_copy(pemb_hbm.at[pl.ds(s0 + ci * _C, _C)], pos_v, psem)

    def wait_loads(buf):
        rows_v, pos_v, _, gsem, psem, _ = buf
        pltpu.make_async_copy(wemb_hbm.at[idx_all.at[pl.ds(0, _C)]],
                              rows_v, gsem).wait()
        pltpu.make_async_copy(pemb_hbm.at[pl.ds(0, _C)], pos_v, psem).wait()

    def issue_out(ci, buf):
        _, _, o_v, _, _, osem = buf
        pltpu.async_copy(o_v, out_hbm.at[pl.ds(base + ci * _C, _C)], osem)

    def wait_out(buf):
        _, _, o_v, _, _, osem = buf
        pltpu.make_async_copy(o_v, out_hbm.at[pl.ds(0, _C)], osem).wait()

    def compute(buf):
        rows_v, pos_v, o_v, _, _, _ = buf

        @plsc.parallel_loop(0, _C, 1, unroll=2)
        def _tok(j):
            sv = jnp.zeros((16,), jnp.float32)
            qv = jnp.zeros((16,), jnp.float32)
            for g in range(_G):
                sl = pl.ds(g * 16, 16)
                x = rows_v[j, sl] + pos_v[j, sl]
                rows_v[j, sl] = x
                sv = sv + x
                qv = qv + x * x
            mean = jnp.sum(sv) * (1.0 / _HID)
            var = jnp.sum(qv) * (1.0 / _HID) - mean * mean
            v16 = jnp.full((16,), 0.0, jnp.float32) + (var + _EPS)
            rstd = _rsqrt16(v16)
            for g0 in range(0, _G, _SB):
                gs = range(g0, min(g0 + _SB, _G))
                sls = [pl.ds(g * 16, 16) for g in gs]
                gms = [gamma_v[sl] for sl in sls]
                bts = [beta_v[sl] for sl in sls]
                xs = [rows_v[j, sl] for sl in sls]
                outs = [(x - mean) * rstd * gm + bt
                        for x, gm, bt in zip(xs, gms, bts)]
                for sl, o in zip(sls, outs):
                    o_v[j, sl] = o

    issue_loads(0, bufs[0])
    issue_loads(1, bufs[1])
    pltpu.make_async_copy(gamma_hbm, gamma_v, prosem).wait()
    pltpu.make_async_copy(beta_hbm, beta_v, prosem).wait()
    n2 = _NCHUNK // 2

    def pipe_body(i, carry):
        @pl.when(i > 0)
        def _():
            wait_out(bufs[0])

        ci0 = i * 2
        wait_loads(bufs[0])
        compute(bufs[0])
        issue_out(ci0, bufs[0])

        @pl.when(i < n2 - 1)
        def _():
            issue_loads(ci0 + 2, bufs[0])

        @pl.when(i > 0)
        def _():
            wait_out(bufs[1])

        wait_loads(bufs[1])
        compute(bufs[1])
        issue_out(ci0 + 1, bufs[1])

        @pl.when(i < n2 - 1)
        def _():
            issue_loads(ci0 + 3, bufs[1])

        return carry

    lax.fori_loop(0, n2, pipe_body, 0)
    wait_out(bufs[0])
    wait_out(bufs[1])


@jax.jit
def _embed_ln(ids_flat, word_emb, pos_emb, gamma, beta):
    mesh = plsc.VectorSubcoreMesh(core_axis_name="c", subcore_axis_name="s")
    kern = functools.partial(
        pl.kernel,
        mesh=mesh,
        out_type=jax.ShapeDtypeStruct((_NT, _HID), jnp.float32),
        scratch_types=[
            pltpu.VMEM((_TPW,), jnp.int32),
            pltpu.VMEM((_C, _HID), jnp.float32),
            pltpu.VMEM((_C, _HID), jnp.float32),
            pltpu.VMEM((_C, _HID), jnp.float32),
            pltpu.VMEM((_C, _HID), jnp.float32),
            pltpu.VMEM((_C, _HID), jnp.float32),
            pltpu.VMEM((_C, _HID), jnp.float32),
            pltpu.VMEM((_HID,), jnp.float32),
            pltpu.VMEM((_HID,), jnp.float32),
            pltpu.SemaphoreType.DMA,
            pltpu.SemaphoreType.DMA,
            pltpu.SemaphoreType.DMA,
            pltpu.SemaphoreType.DMA,
            pltpu.SemaphoreType.DMA,
            pltpu.SemaphoreType.DMA,
            pltpu.SemaphoreType.DMA,
        ],
        compiler_params=pltpu.CompilerParams(needs_layout_passes=False),
    )(_tile_body)
    return kern(ids_flat, word_emb, pos_emb, gamma, beta)


def kernel(input_ids, word_emb, pos_emb, gamma, beta):
    ids_flat = input_ids.reshape(-1).astype(jnp.int32)
    out = _embed_ln(ids_flat, word_emb, pos_emb, gamma, beta)
    return out.reshape(_B, _S, _HID)


# speedup vs baseline: 1.0284x; 1.0284x over previous
import functools

import jax
import jax.numpy as jnp
from jax import lax
from jax.experimental import pallas as pl
from jax.experimental.pallas import tpu as pltpu
from jax.experimental.pallas import tpu_sc as plsc

_VOCAB = 32000
_HID = 768
_B = 4
_S = 2048
_EPS = 1e-12
_NT = _B * _S
_NW = 32
_TPW = _NT // _NW
_C = 16
_NCHUNK = _TPW // _C
_G = _HID // 16
_SB = 12


def _rsqrt16(v):
    i = plsc.bitcast(v, jnp.int32)
    y = plsc.bitcast(jnp.int32(0x5F3759DF) - (i >> 1), jnp.float32)
    h = v * 0.5
    y = y * (1.5 - h * y * y)
    y = y * (1.5 - h * y * y)
    return y


def _tile_body(ids_hbm, wemb_hbm, pemb_hbm, gamma_hbm, beta_hbm, out_hbm,
               idx_all, rows_a, rows_b, pos_a, pos_b, o_a, o_b,
               gamma_v, beta_v, gsem_a, gsem_b, psem_a, psem_b,
               osem_a, osem_b, prosem):
    wid = lax.axis_index("s") * 2 + lax.axis_index("c")
    base = wid * _TPW
    s0 = base % _S

    pltpu.sync_copy(ids_hbm.at[pl.ds(base, _TPW)], idx_all)
    pltpu.async_copy(gamma_hbm, gamma_v, prosem)
    pltpu.async_copy(beta_hbm, beta_v, prosem)

    bufs = (
        (rows_a, pos_a, o_a, gsem_a, psem_a, osem_a),
        (rows_b, pos_b, o_b, gsem_b, psem_b, osem_b),
    )

    def issue_loads(ci, buf):
        rows_v, pos_v, _, gsem, psem, _ = buf
        pltpu.async_copy(wemb_hbm.at[idx_all.at[pl.ds(ci * _C, _C)]],
                         rows_v, gsem)
        pltpu.async_copy(pemb_hbm.at[pl.ds(s0 + ci * _C, _C)], pos_v, psem)

    def wait_loads(buf):
        rows_v, pos_v, _, gsem, psem, _ = buf
        pltpu.make_async_copy(wemb_hbm.at[idx_all.at[pl.ds(0, _C)]],
                              rows_v, gsem).wait()
        pltpu.make_async_copy(pemb_hbm.at[pl.ds(0, _C)], pos_v, psem).wait()

    def issue_out(ci, buf):
        _, _, o_v, _, _, osem = buf
        pltpu.async_copy(o_v, out_hbm.at[pl.ds(base + ci * _C, _C)], osem)

    def wait_out(buf):
        _, _, o_v, _, _, osem = buf
        pltpu.make_async_copy(o_v, out_hbm.at[pl.ds(0, _C)], osem).wait()

    def compute(buf):
        rows_v, pos_v, o_v, _, _, _ = buf

        @plsc.parallel_loop(0, _C, 1, unroll=2)
        def _tok(j):
            sv = jnp.zeros((16,), jnp.float32)
            qv = jnp.zeros((16,), jnp.float32)
            for g in range(_G):
                sl = pl.ds(g * 16, 16)
                x = rows_v[j, sl] + pos_v[j, sl]
                rows_v[j, sl] = x
                sv = sv + x
                qv = qv + x * x
            mean = jnp.sum(sv) * (1.0 / _HID)
            var = jnp.sum(qv) * (1.0 / _HID) - mean * mean
            v16 = jnp.full((16,), 0.0, jnp.float32) + (var + _EPS)
            rstd = _rsqrt16(v16)
            for g0 in range(0, _G, _SB):
                gs = range(g0, min(g0 + _SB, _G))
                sls = [pl.ds(g * 16, 16) for g in gs]
                gms = [gamma_v[sl] for sl in sls]
                bts = [beta_v[sl] for sl in sls]
                xs = [rows_v[j, sl] for sl in sls]
                outs = [(x - mean) * rstd * gm + bt
                        for x, gm, bt in zip(xs, gms, bts)]
                for sl, o in zip(sls, outs):
                    o_v[j, sl] = o

    issue_loads(0, bufs[0])
    issue_loads(1, bufs[1])
    pltpu.make_async_copy(gamma_hbm, gamma_v, prosem).wait()
    pltpu.make_async_copy(beta_hbm, beta_v, prosem).wait()
    n2 = _NCHUNK // 2

    def pipe_body(i, carry):
        @pl.when(i > 0)
        def _():
            wait_out(bufs[0])

        ci0 = i * 2
        wait_loads(bufs[0])
        compute(bufs[0])
        issue_out(ci0, bufs[0])

        @pl.when(i < n2 - 1)
        def _():
            issue_loads(ci0 + 2, bufs[0])

        @pl.when(i > 0)
        def _():
            wait_out(bufs[1])

        wait_loads(bufs[1])
        compute(bufs[1])
        issue_out(ci0 + 1, bufs[1])

        @pl.when(i < n2 - 1)
        def _():
            issue_loads(ci0 + 3, bufs[1])

        return carry

    lax.fori_loop(0, n2, pipe_body, 0)
    wait_out(bufs[0])
    wait_out(bufs[1])


@jax.jit
def _embed_ln(ids_flat, word_emb, pos_emb, gamma, beta):
    mesh = plsc.VectorSubcoreMesh(core_axis_name="c", subcore_axis_name="s")
    kern = functools.partial(
        pl.kernel,
        mesh=mesh,
        out_type=jax.ShapeDtypeStruct((_NT, _HID), jnp.float32),
        scratch_types=[
            pltpu.VMEM((_TPW,), jnp.int32),
            pltpu.VMEM((_C, _HID), jnp.float32),
            pltpu.VMEM((_C, _HID), jnp.float32),
            pltpu.VMEM((_C, _HID), jnp.float32),
            pltpu.VMEM((_C, _HID), jnp.float32),
            pltpu.VMEM((_C, _HID), jnp.float32),
            pltpu.VMEM((_C, _HID), jnp.float32),
            pltpu.VMEM((_HID,), jnp.float32),
            pltpu.VMEM((_HID,), jnp.float32),
            pltpu.SemaphoreType.DMA,
            pltpu.SemaphoreType.DMA,
            pltpu.SemaphoreType.DMA,
            pltpu.SemaphoreType.DMA,
            pltpu.SemaphoreType.DMA,
            pltpu.SemaphoreType.DMA,
            pltpu.SemaphoreType.DMA,
        ],
        compiler_params=pltpu.CompilerParams(needs_layout_passes=False),
    )(_tile_body)
    return kern(ids_flat, word_emb, pos_emb, gamma, beta)


def kernel(input_ids, word_emb, pos_emb, gamma, beta):
    ids_flat = input_ids.reshape(-1).astype(jnp.int32)
    out = _embed_ln(ids_flat, word_emb, pos_emb, gamma, beta)
    return out.reshape(_B, _S, _HID)
